# Initial kernel scaffold; baseline (speedup 1.0000x reference)
import jax
import jax.numpy as jnp
from jax.experimental import pallas as pl


def kernel(x, W):
    raise NotImplementedError("write your pallas kernel here")


# speedup vs baseline: 2.4982x; 2.4982x over previous
import functools

import jax
import jax.numpy as jnp
import numpy as np
from jax import lax
from jax.experimental import pallas as pl
from jax.experimental.pallas import tpu as pltpu
from jax.experimental.pallas import tpu_sc as plsc

_VOCAB = 100000
_D = 64
_SEQ = 200
_BATCH = 4096
_SCALE = float(np.sqrt(_D))

_NC = 2
_NS = 16
_NW = _NC * _NS

_ROWS = _BATCH * _SEQ
_ROWS_W = _ROWS // _NW
_GW = 128
_JPW = _ROWS_W // _GW
_JPC = 5
_CHUNK = _JPC * _GW
_NCHUNK = _ROWS_W // _CHUNK


def _pos_encoding() -> jnp.ndarray:
    position = np.arange(_SEQ)[:, np.newaxis]
    div_term = np.exp(np.arange(0, _D, 2) * -(np.log(10000.0) / _D))
    pe = np.zeros((_SEQ, _D), dtype=np.float32)
    pe[:, 0::2] = np.sin(position * div_term)
    pe[:, 1::2] = np.cos(position * div_term)
    return jnp.asarray(pe)


def _sc_kernel(x_hbm, pe_hbm, w_hbm, out_hbm, idx_v, pe_v, buf_v, gsem):
    wid = lax.axis_index("s") * _NC + lax.axis_index("c")

    pltpu.sync_copy(x_hbm.at[wid], idx_v)
    pltpu.sync_copy(pe_hbm, pe_v)

    def chunk_body(c, _):
        j0 = c * _JPC
        cps = []
        for jj in range(_JPC):
            cps.append(pltpu.async_copy(
                w_hbm.at[idx_v.at[j0 + jj]],
                buf_v.at[pl.ds(jj * _GW, _GW)],
                gsem,
            ))
        for cp in cps:
            cp.wait()

        s0 = lax.rem(c * _CHUNK, _SEQ)

        def row_body(r, s):
            for d in range(_D // 16):
                sl = pl.ds(d * 16, 16)
                buf_v[r, sl] = buf_v[r, sl] * _SCALE + pe_v[s, sl]
            s = s + 1
            return jnp.where(s == _SEQ, 0, s)

        lax.fori_loop(0, _CHUNK, row_body, s0)

        pltpu.sync_copy(
            buf_v,
            out_hbm.at[pl.ds(wid * _ROWS_W + c * _CHUNK, _CHUNK)],
        )
        return _

    lax.fori_loop(0, _NCHUNK, chunk_body, 0)


@jax.jit
def _run(x3, pe, W):
    mesh = plsc.VectorSubcoreMesh(core_axis_name="c", subcore_axis_name="s")
    f = functools.partial(
        pl.kernel,
        mesh=mesh,
        out_type=jax.ShapeDtypeStruct((_ROWS, _D), jnp.float32),
        scratch_types=[
            pltpu.VMEM((_JPW, _GW), jnp.int32),
            pltpu.VMEM((_SEQ, _D), jnp.float32),
            pltpu.VMEM((_CHUNK, _D), jnp.float32),
            pltpu.SemaphoreType.DMA,
        ],
        compiler_params=pltpu.CompilerParams(use_tc_tiling_on_sc=False),
    )(_sc_kernel)
    return f(x3, pe, W)


def kernel(x, W):
    x3 = x.reshape(_NW, _JPW, _GW)
    out = _run(x3, _pos_encoding(), W)
    return out.reshape(_BATCH, _SEQ, _D)


# speedup vs baseline: 2.7282x; 1.0921x over previous
import functools

import jax
import jax.numpy as jnp
import numpy as np
from jax import lax
from jax.experimental import pallas as pl
from jax.experimental.pallas import tpu as pltpu
from jax.experimental.pallas import tpu_sc as plsc

_VOCAB = 100000
_D = 64
_SEQ = 200
_BATCH = 4096
_SCALE = float(np.sqrt(_D))

_NC = 2
_NS = 16
_NW = _NC * _NS

_ROWS = _BATCH * _SEQ
_ROWS_W = _ROWS // _NW
_GW = 128
_JPW = _ROWS_W // _GW
_JPC = 5
_CHUNK = _JPC * _GW
_NCHUNK = _ROWS_W // _CHUNK


def _pos_encoding() -> jnp.ndarray:
    position = np.arange(_SEQ)[:, np.newaxis]
    div_term = np.exp(np.arange(0, _D, 2) * -(np.log(10000.0) / _D))
    pe = np.zeros((_SEQ, _D), dtype=np.float32)
    pe[:, 0::2] = np.sin(position * div_term)
    pe[:, 1::2] = np.cos(position * div_term)
    return jnp.asarray(pe)


def _sc_kernel(x_hbm, pe_hbm, w_hbm, out_hbm, idx_v, pe_v, buf_a, buf_b,
               gsem_a, gsem_b, ssem_a, ssem_b):
    wid = lax.axis_index("s") * _NC + lax.axis_index("c")

    pltpu.sync_copy(x_hbm.at[wid], idx_v)
    pltpu.sync_copy(pe_hbm, pe_v)

    def fire_gather(c, buf, gsem):
        j0 = c * _JPC
        for jj in range(_JPC):
            pltpu.async_copy(
                w_hbm.at[idx_v.at[j0 + jj]],
                buf.at[pl.ds(jj * _GW, _GW)],
                gsem,
            )

    def drain(buf, sem):
        pltpu.make_async_copy(out_hbm.at[pl.ds(0, _CHUNK)], buf, sem).wait()

    def fire_store(c, buf, ssem):
        pltpu.async_copy(
            buf, out_hbm.at[pl.ds(wid * _ROWS_W + c * _CHUNK, _CHUNK)], ssem)

    def compute(c, buf):
        s0 = lax.rem(c * _CHUNK, _SEQ)

        def row_body(i, s):
            for u in range(2):
                r = i * 2 + u
                for d in range(_D // 16):
                    sl = pl.ds(d * 16, 16)
                    buf[r, sl] = buf[r, sl] * _SCALE + pe_v[s, sl]
                s = s + 1
                s = jnp.where(s == _SEQ, 0, s)
            return s

        lax.fori_loop(0, _CHUNK // 2, row_body, s0)

    fire_gather(0, buf_a, gsem_a)
    fire_gather(1, buf_b, gsem_b)

    def pair_body(i, _):
        c0 = i * 2
        drain(buf_a, gsem_a)
        compute(c0, buf_a)
        fire_store(c0, buf_a, ssem_a)
        drain(buf_b, gsem_b)
        compute(c0 + 1, buf_b)
        fire_store(c0 + 1, buf_b, ssem_b)
        @pl.when(i + 1 < _NCHUNK // 2)
        def _refill():
            drain(buf_a, ssem_a)
            fire_gather(c0 + 2, buf_a, gsem_a)
            drain(buf_b, ssem_b)
            fire_gather(c0 + 3, buf_b, gsem_b)
        return _

    lax.fori_loop(0, _NCHUNK // 2, pair_body, 0)

    drain(buf_a, ssem_a)
    drain(buf_b, ssem_b)


@jax.jit
def _run(x3, pe, W):
    mesh = plsc.VectorSubcoreMesh(core_axis_name="c", subcore_axis_name="s")
    f = functools.partial(
        pl.kernel,
        mesh=mesh,
        out_type=jax.ShapeDtypeStruct((_ROWS, _D), jnp.float32),
        scratch_types=[
            pltpu.VMEM((_JPW, _GW), jnp.int32),
            pltpu.VMEM((_SEQ, _D), jnp.float32),
            pltpu.VMEM((_CHUNK, _D), jnp.float32),
            pltpu.VMEM((_CHUNK, _D), jnp.float32),
            pltpu.SemaphoreType.DMA,
            pltpu.SemaphoreType.DMA,
            pltpu.SemaphoreType.DMA,
            pltpu.SemaphoreType.DMA,
        ],
        compiler_params=pltpu.CompilerParams(use_tc_tiling_on_sc=False),
    )(_sc_kernel)
    return f(x3, pe, W)


def kernel(x, W):
    x3 = x.reshape(_NW, _JPW, _GW)
    out = _run(x3, _pos_encoding(), W)
    return out.reshape(_BATCH, _SEQ, _D)
